# baseline (speedup 1.0000x reference)
import jax
import jax.numpy as jnp
from jax.experimental import pallas as pl

_EPS = 1e-5
_HEAD_OUT = (2, 1, 3, 2, 2, 10)
_L = 200
_CIN = 128
_CH = 64


def _fused_heads_kernel(x_ref, *refs):
    x = x_ref[...]
    n = len(_HEAD_OUT)
    out_refs = refs[5 * n:]
    for i in range(n):
        w0_ref, g_ref, b_ref, w1_ref, b1_ref = refs[5 * i:5 * i + 5]
        h = jnp.dot(w0_ref[...], x, preferred_element_type=jnp.float32)
        mean = jnp.mean(h, axis=1, keepdims=True)
        centered = h - mean
        var = jnp.mean(centered * centered, axis=1, keepdims=True)
        hn = centered * jax.lax.rsqrt(var + _EPS)
        hn = hn * g_ref[...] + b_ref[...]
        hn = jnp.maximum(hn, 0.0)
        out = jnp.dot(w1_ref[...], hn, preferred_element_type=jnp.float32)
        out_refs[i][...] = out + b1_ref[...]


def kernel(x, center_w0, center_bn_gamma, center_bn_beta, center_w1, center_b1,
           height_w0, height_bn_gamma, height_bn_beta, height_w1, height_b1,
           dim_w0, dim_bn_gamma, dim_bn_beta, dim_w1, dim_b1,
           rot_w0, rot_bn_gamma, rot_bn_beta, rot_w1, rot_b1,
           vel_w0, vel_bn_gamma, vel_bn_beta, vel_w1, vel_b1,
           heatmap_w0, heatmap_bn_gamma, heatmap_bn_beta, heatmap_w1, heatmap_b1):
    heads = [
        (center_w0, center_bn_gamma, center_bn_beta, center_w1, center_b1),
        (height_w0, height_bn_gamma, height_bn_beta, height_w1, height_b1),
        (dim_w0, dim_bn_gamma, dim_bn_beta, dim_w1, dim_b1),
        (rot_w0, rot_bn_gamma, rot_bn_beta, rot_w1, rot_b1),
        (vel_w0, vel_bn_gamma, vel_bn_beta, vel_w1, vel_b1),
        (heatmap_w0, heatmap_bn_gamma, heatmap_bn_beta, heatmap_w1, heatmap_b1),
    ]
    args = [x.reshape(_CIN, _L)]
    for w0, g, b, w1, b1 in heads:
        args += [w0, g.reshape(_CH, 1), b.reshape(_CH, 1),
                 w1, b1.reshape(-1, 1)]
    outs = pl.pallas_call(
        _fused_heads_kernel,
        out_shape=tuple(
            jax.ShapeDtypeStruct((oc, _L), jnp.float32) for oc in _HEAD_OUT
        ),
    )(*args)
    return tuple(o.reshape(1, oc, _L) for o, oc in zip(outs, _HEAD_OUT))
